# baseline (speedup 1.0000x reference)
import functools

import jax
import jax.numpy as jnp
from jax import lax
from jax.experimental import pallas as pl
from jax.experimental.pallas import tpu as pltpu
from jax.experimental.pallas import tpu_sc as plsc

N = 10000
D = 128
E = 320000

NC = 2
NS = 16
NW = NC * NS
CHUNK = 128
N_CHUNKS = 80
EPW = N_CHUNKS * CHUNK
E_PAD = NW * EPW
N_PAD = 10112
ROWS_PW = N_PAD // NS

_mesh = plsc.VectorSubcoreMesh(
    core_axis_name="c", subcore_axis_name="s", num_cores=NC, num_subcores=NS
)


@functools.partial(
    pl.kernel,
    out_type=jax.ShapeDtypeStruct((NC, N_PAD, D), jnp.float32),
    mesh=_mesh,
    scratch_types=[
        pltpu.VMEM((N_CHUNKS, CHUNK), jnp.int32),
        pltpu.VMEM((CHUNK, D), jnp.float32),
        pltpu.VMEM_SHARED((N_PAD, D), jnp.float32),
    ],
)
def _deg_kernel(dst_hbm, ones_hbm, zeros_hbm, out_hbm, idx_v, ones_v, deg_sh):
    c = lax.axis_index("c")
    s = lax.axis_index("s")
    wid = s * NC + c

    pltpu.sync_copy(ones_hbm, ones_v)
    pltpu.sync_copy(
        zeros_hbm.at[pl.ds(s * ROWS_PW, ROWS_PW)],
        deg_sh.at[pl.ds(s * ROWS_PW, ROWS_PW)],
    )
    pltpu.sync_copy(dst_hbm.at[wid], idx_v)
    plsc.subcore_barrier()

    def _body(j, carry):
        pltpu.sync_copy(ones_v, deg_sh.at[idx_v.at[j]], add=True)
        return carry

    lax.fori_loop(0, N_CHUNKS, _body, 0)
    plsc.subcore_barrier()

    pltpu.sync_copy(
        deg_sh.at[pl.ds(s * ROWS_PW, ROWS_PW)],
        out_hbm.at[c, pl.ds(s * ROWS_PW, ROWS_PW)],
    )


GCH = 64
NBUF = 4
WIN = 32
C0_WIN = 6
C1_WIN = 4
C0_CH = C0_WIN * WIN
C1_CH = C1_WIN * WIN
E_SPLIT = NS * C0_CH * GCH


@functools.partial(
    pl.kernel,
    out_type=jax.ShapeDtypeStruct((NC, N_PAD, D), jnp.float32),
    mesh=_mesh,
    scratch_types=[
        pltpu.VMEM((WIN, GCH), jnp.int32),
        pltpu.VMEM((WIN, GCH), jnp.int32),
        pltpu.VMEM((GCH, D), jnp.float32),
        pltpu.VMEM((GCH, D), jnp.float32),
        pltpu.VMEM((GCH, D), jnp.float32),
        pltpu.VMEM((GCH, D), jnp.float32),
        pltpu.VMEM_SHARED((N_PAD, D), jnp.float32),
        pltpu.SemaphoreType.DMA,
        pltpu.SemaphoreType.DMA,
    ],
)
def _agg_kernel(y_hbm, src0_hbm, dst0_hbm, src1_hbm, dst1_hbm, zeros_hbm,
                out_hbm, srcv, dstv, r0, r1, r2, r3, agg_sh, gsem, ssem):
    c = lax.axis_index("c")
    s = lax.axis_index("s")
    rows = (r0, r1, r2, r3)

    pltpu.sync_copy(
        zeros_hbm.at[pl.ds(s * ROWS_PW, ROWS_PW)],
        agg_sh.at[pl.ds(s * ROWS_PW, ROWS_PW)],
    )
    plsc.subcore_barrier()

    def _window(src_arr, dst_arr, h):
        pltpu.sync_copy(src_arr.at[s, pl.ds(h * WIN, WIN)], srcv)
        pltpu.sync_copy(dst_arr.at[s, pl.ds(h * WIN, WIN)], dstv)
        for b in range(NBUF - 1):
            pltpu.async_copy(y_hbm.at[srcv.at[b]], rows[b], gsem)

        def _group(g, carry):
            for b in range(NBUF):
                j = g * NBUF + b
                bp = (b + NBUF - 1) % NBUF
                pltpu.make_async_copy(
                    y_hbm.at[srcv.at[j]], rows[b], gsem
                ).wait()

                @pl.when(j > 0)
                def _():
                    pltpu.make_async_copy(
                        rows[bp], agg_sh.at[dstv.at[j - 1]], ssem
                    ).wait()

                @pl.when(j + NBUF - 1 < WIN)
                def _():
                    pltpu.async_copy(
                        y_hbm.at[srcv.at[j + NBUF - 1]], rows[bp], gsem
                    )

                pltpu.async_copy(
                    rows[b], agg_sh.at[dstv.at[j]], ssem, add=True
                )
            return carry

        lax.fori_loop(0, WIN // NBUF, _group, 0)
        pltpu.make_async_copy(
            rows[(WIN - 1) % NBUF], agg_sh.at[dstv.at[WIN - 1]], ssem
        ).wait()

    @pl.when(c == 0)
    def _():
        for h in range(C0_WIN):
            _window(src0_hbm, dst0_hbm, h)

    @pl.when(c == 1)
    def _():
        for h in range(C1_WIN):
            _window(src1_hbm, dst1_hbm, h)

    plsc.subcore_barrier()

    pltpu.sync_copy(
        agg_sh.at[pl.ds(s * ROWS_PW, ROWS_PW)],
        out_hbm.at[c, pl.ds(s * ROWS_PW, ROWS_PW)],
    )


_R = 1000


def _prep_body(dga_ref, dgb_ref, x_ref, y_ref):
    deg = dga_ref[...] + dgb_ref[...] + 1.0
    dis = lax.rsqrt(deg)
    y_ref[...] = x_ref[...] * dis


def _out_body(dga_ref, dgb_ref, y_ref, p_ref, w_ref, b_ref, o_ref):
    deg = dga_ref[...] + dgb_ref[...] + 1.0
    dis = lax.rsqrt(deg)
    t = y_ref[...] + p_ref[0] + p_ref[1]
    acc = jnp.dot(t, w_ref[...], preferred_element_type=jnp.float32)
    acc = acc * dis + b_ref[...]
    o_ref[...] = jnp.where(acc > 0, acc, 0.01 * acc)


def kernel(X, A, W, b):
    src = A[0].astype(jnp.int32)
    dst = A[1].astype(jnp.int32)
    pad = jnp.full((E_PAD - E,), N, dtype=jnp.int32)
    src_f = jnp.concatenate([src, pad])
    dst_f = jnp.concatenate([dst, pad])
    dst_p = dst_f.reshape(NW, N_CHUNKS, CHUNK)
    src0 = src_f[:E_SPLIT].reshape(NS, C0_CH, GCH)
    dst0 = dst_f[:E_SPLIT].reshape(NS, C0_CH, GCH)
    src1 = src_f[E_SPLIT:].reshape(NS, C1_CH, GCH)
    dst1 = dst_f[E_SPLIT:].reshape(NS, C1_CH, GCH)

    zerosD = jnp.zeros((N_PAD, D), jnp.float32)
    onesD = jnp.ones((CHUNK, D), jnp.float32)

    degp = _deg_kernel(dst_p, onesD, zerosD)
    dga = degp[0, :, :1]
    dgb = degp[1, :, :1]

    y = pl.pallas_call(
        _prep_body,
        grid=(N // _R,),
        in_specs=[
            pl.BlockSpec((_R, 1), lambda i: (i, 0)),
            pl.BlockSpec((_R, 1), lambda i: (i, 0)),
            pl.BlockSpec((_R, D), lambda i: (i, 0)),
        ],
        out_specs=pl.BlockSpec((_R, D), lambda i: (i, 0)),
        out_shape=jax.ShapeDtypeStruct((N_PAD, D), jnp.float32),
    )(dga, dgb, X)

    partials = _agg_kernel(y, src0, dst0, src1, dst1, zerosD)

    out = pl.pallas_call(
        _out_body,
        grid=(N // _R,),
        in_specs=[
            pl.BlockSpec((_R, 1), lambda i: (i, 0)),
            pl.BlockSpec((_R, 1), lambda i: (i, 0)),
            pl.BlockSpec((_R, D), lambda i: (i, 0)),
            pl.BlockSpec((2, _R, D), lambda i: (0, i, 0)),
            pl.BlockSpec((D, D), lambda i: (0, 0)),
            pl.BlockSpec((1, D), lambda i: (0, 0)),
        ],
        out_specs=pl.BlockSpec((_R, D), lambda i: (i, 0)),
        out_shape=jax.ShapeDtypeStruct((N, D), jnp.float32),
    )(dga, dgb, y, partials, W, b.reshape(1, D))

    return out


# speedup vs baseline: 1.1554x; 1.1554x over previous
import functools

import jax
import jax.numpy as jnp
from jax import lax
from jax.experimental import pallas as pl
from jax.experimental.pallas import tpu as pltpu
from jax.experimental.pallas import tpu_sc as plsc

N = 10000
D = 128
E = 320000

NC = 2
NS = 16
NW = NC * NS
CHUNK = 128
N_CHUNKS = 80
EPW = N_CHUNKS * CHUNK
E_PAD = NW * EPW
N_PAD = 10112
ROWS_PW = N_PAD // NS

_mesh = plsc.VectorSubcoreMesh(
    core_axis_name="c", subcore_axis_name="s", num_cores=NC, num_subcores=NS
)


@functools.partial(
    pl.kernel,
    out_type=jax.ShapeDtypeStruct((NC, N_PAD, D), jnp.float32),
    mesh=_mesh,
    scratch_types=[
        pltpu.VMEM((N_CHUNKS, CHUNK), jnp.int32),
        pltpu.VMEM((CHUNK, D), jnp.float32),
        pltpu.VMEM_SHARED((N_PAD, D), jnp.float32),
        pltpu.SemaphoreType.DMA,
    ],
)
def _deg_kernel(dst_hbm, ones_hbm, zeros_hbm, out_hbm, idx_v, ones_v, deg_sh,
                dsem):
    c = lax.axis_index("c")
    s = lax.axis_index("s")
    wid = s * NC + c

    pltpu.sync_copy(ones_hbm, ones_v)
    pltpu.sync_copy(
        zeros_hbm.at[pl.ds(s * ROWS_PW, ROWS_PW)],
        deg_sh.at[pl.ds(s * ROWS_PW, ROWS_PW)],
    )
    pltpu.sync_copy(dst_hbm.at[wid], idx_v)
    plsc.subcore_barrier()

    def _body(j, carry):
        pltpu.async_copy(ones_v, deg_sh.at[idx_v.at[j]], dsem, add=True)
        return carry

    lax.fori_loop(0, N_CHUNKS, _body, 0)

    def _drain(j, carry):
        pltpu.make_async_copy(ones_v, deg_sh.at[idx_v.at[j]], dsem).wait()
        return carry

    lax.fori_loop(0, N_CHUNKS, _drain, 0)
    plsc.subcore_barrier()

    pltpu.sync_copy(
        deg_sh.at[pl.ds(s * ROWS_PW, ROWS_PW)],
        out_hbm.at[c, pl.ds(s * ROWS_PW, ROWS_PW)],
    )


GCH = 128
NBUF = 2
WIN = 40
C0_WIN = 3
C1_WIN = 1
C0_CH = C0_WIN * WIN
C1_CH = C1_WIN * WIN
E_SPLIT = NS * C0_CH * GCH


@functools.partial(
    pl.kernel,
    out_type=jax.ShapeDtypeStruct((NC, N_PAD, D), jnp.float32),
    mesh=_mesh,
    scratch_types=[
        pltpu.VMEM((WIN, GCH), jnp.int32),
        pltpu.VMEM((WIN, GCH), jnp.int32),
        pltpu.VMEM((GCH, D), jnp.float32),
        pltpu.VMEM((GCH, D), jnp.float32),
        pltpu.VMEM_SHARED((N_PAD, D), jnp.float32),
        pltpu.SemaphoreType.DMA,
        pltpu.SemaphoreType.DMA,
    ],
)
def _agg_kernel(y_hbm, src0_hbm, dst0_hbm, src1_hbm, dst1_hbm, zeros_hbm,
                out_hbm, srcv, dstv, r0, r1, agg_sh, gsem, ssem):
    c = lax.axis_index("c")
    s = lax.axis_index("s")
    rows = (r0, r1)

    pltpu.sync_copy(
        zeros_hbm.at[pl.ds(s * ROWS_PW, ROWS_PW)],
        agg_sh.at[pl.ds(s * ROWS_PW, ROWS_PW)],
    )
    plsc.subcore_barrier()

    def _window(src_arr, dst_arr, h):
        pltpu.sync_copy(src_arr.at[s, pl.ds(h * WIN, WIN)], srcv)
        pltpu.sync_copy(dst_arr.at[s, pl.ds(h * WIN, WIN)], dstv)
        for b in range(NBUF - 1):
            pltpu.async_copy(y_hbm.at[srcv.at[b]], rows[b], gsem)

        def _group(g, carry):
            for b in range(NBUF):
                j = g * NBUF + b
                bp = (b + NBUF - 1) % NBUF
                pltpu.make_async_copy(
                    y_hbm.at[srcv.at[j]], rows[b], gsem
                ).wait()

                @pl.when(j > 0)
                def _():
                    pltpu.make_async_copy(
                        rows[bp], agg_sh.at[dstv.at[j - 1]], ssem
                    ).wait()

                @pl.when(j + NBUF - 1 < WIN)
                def _():
                    pltpu.async_copy(
                        y_hbm.at[srcv.at[j + NBUF - 1]], rows[bp], gsem
                    )

                pltpu.async_copy(
                    rows[b], agg_sh.at[dstv.at[j]], ssem, add=True
                )
            return carry

        lax.fori_loop(0, WIN // NBUF, _group, 0)
        pltpu.make_async_copy(
            rows[(WIN - 1) % NBUF], agg_sh.at[dstv.at[WIN - 1]], ssem
        ).wait()

    @pl.when(c == 0)
    def _():
        for h in range(C0_WIN):
            _window(src0_hbm, dst0_hbm, h)

    @pl.when(c == 1)
    def _():
        for h in range(C1_WIN):
            _window(src1_hbm, dst1_hbm, h)

    plsc.subcore_barrier()

    pltpu.sync_copy(
        agg_sh.at[pl.ds(s * ROWS_PW, ROWS_PW)],
        out_hbm.at[c, pl.ds(s * ROWS_PW, ROWS_PW)],
    )


_R = 1000


def _prep_body(dga_ref, dgb_ref, x_ref, y_ref):
    deg = dga_ref[...] + dgb_ref[...] + 1.0
    dis = lax.rsqrt(deg)
    y_ref[...] = x_ref[...] * dis


def _out_body(dga_ref, dgb_ref, y_ref, p_ref, w_ref, b_ref, o_ref):
    deg = dga_ref[...] + dgb_ref[...] + 1.0
    dis = lax.rsqrt(deg)
    t = y_ref[...] + p_ref[0] + p_ref[1]
    acc = jnp.dot(t, w_ref[...], preferred_element_type=jnp.float32)
    acc = acc * dis + b_ref[...]
    o_ref[...] = jnp.where(acc > 0, acc, 0.01 * acc)


def kernel(X, A, W, b):
    src = A[0].astype(jnp.int32)
    dst = A[1].astype(jnp.int32)
    pad = jnp.full((E_PAD - E,), N, dtype=jnp.int32)
    src_f = jnp.concatenate([src, pad])
    dst_f = jnp.concatenate([dst, pad])
    dst_p = dst_f.reshape(NW, N_CHUNKS, CHUNK)
    src0 = src_f[:E_SPLIT].reshape(NS, C0_CH, GCH)
    dst0 = dst_f[:E_SPLIT].reshape(NS, C0_CH, GCH)
    src1 = src_f[E_SPLIT:].reshape(NS, C1_CH, GCH)
    dst1 = dst_f[E_SPLIT:].reshape(NS, C1_CH, GCH)

    zerosD = jnp.zeros((N_PAD, D), jnp.float32)
    onesD = jnp.ones((CHUNK, D), jnp.float32)

    degp = _deg_kernel(dst_p, onesD, zerosD)
    dga = degp[0, :, :1]
    dgb = degp[1, :, :1]

    y = pl.pallas_call(
        _prep_body,
        grid=(N // _R,),
        in_specs=[
            pl.BlockSpec((_R, 1), lambda i: (i, 0)),
            pl.BlockSpec((_R, 1), lambda i: (i, 0)),
            pl.BlockSpec((_R, D), lambda i: (i, 0)),
        ],
        out_specs=pl.BlockSpec((_R, D), lambda i: (i, 0)),
        out_shape=jax.ShapeDtypeStruct((N_PAD, D), jnp.float32),
    )(dga, dgb, X)

    partials = _agg_kernel(y, src0, dst0, src1, dst1, zerosD)

    out = pl.pallas_call(
        _out_body,
        grid=(N // _R,),
        in_specs=[
            pl.BlockSpec((_R, 1), lambda i: (i, 0)),
            pl.BlockSpec((_R, 1), lambda i: (i, 0)),
            pl.BlockSpec((_R, D), lambda i: (i, 0)),
            pl.BlockSpec((2, _R, D), lambda i: (0, i, 0)),
            pl.BlockSpec((D, D), lambda i: (0, 0)),
            pl.BlockSpec((1, D), lambda i: (0, 0)),
        ],
        out_specs=pl.BlockSpec((_R, D), lambda i: (i, 0)),
        out_shape=jax.ShapeDtypeStruct((N, D), jnp.float32),
    )(dga, dgb, y, partials, W, b.reshape(1, D))

    return out
